# baseline (speedup 1.0000x reference)
import functools

import jax
import jax.numpy as jnp
from jax import lax
from jax.experimental import pallas as pl
from jax.experimental.pallas import tpu as pltpu
from jax.experimental.pallas import tpu_sc as plsc


def _make_gather(V, D, B):
  info = plsc.get_sparse_core_info()
  NC, NS = info.num_cores, info.num_subcores
  NW = NC * NS
  assert B % NW == 0
  b_per_w = B // NW
  CHUNK = 512
  assert b_per_w % (2 * CHUNK) == 0
  n_chunks = b_per_w // CHUNK

  mesh = plsc.VectorSubcoreMesh(core_axis_name="c", subcore_axis_name="s")

  @functools.partial(
      pl.kernel,
      mesh=mesh,
      compiler_params=pltpu.CompilerParams(use_tc_tiling_on_sc=False),
      out_type=jax.ShapeDtypeStruct((B, D), jnp.float32),
      scratch_types=[
          pltpu.VMEM((b_per_w,), jnp.int32),
          pltpu.VMEM((CHUNK, D), jnp.float32),
          pltpu.VMEM((CHUNK, D), jnp.float32),
          pltpu.SemaphoreType.DMA,
          pltpu.SemaphoreType.DMA,
      ],
  )
  def k(x_hbm, table_hbm, out_hbm, idx_v, rows0, rows1, sem0, sem1):
    wid = lax.axis_index("s") * NC + lax.axis_index("c")
    base = wid * b_per_w
    pltpu.sync_copy(x_hbm.at[pl.ds(base, b_per_w)], idx_v)
    rows = (rows0, rows1)
    sems = (sem0, sem1)

    def idx_slice(c):
      return idx_v.at[pl.ds(pl.multiple_of(c * CHUNK, CHUNK), CHUNK)]

    pltpu.async_copy(table_hbm.at[idx_slice(0)], rows0, sem0)

    def step(c, buf):
      nxt = c + 1

      @pl.when(nxt < n_chunks)
      def _():
        pltpu.async_copy(table_hbm.at[idx_slice(nxt)], rows[1 - buf],
                         sems[1 - buf])

      pltpu.make_async_copy(table_hbm.at[idx_slice(c)], rows[buf],
                            sems[buf]).wait()
      pltpu.sync_copy(
          rows[buf],
          out_hbm.at[pl.ds(pl.multiple_of(base + c * CHUNK, CHUNK), CHUNK)])

    def loop_body(i, carry):
      step(2 * i, 0)
      step(2 * i + 1, 1)
      return carry

    lax.fori_loop(0, n_chunks // 2, loop_body, 0)

  return k


_gather_cache = {}


def kernel(x, weight):
  V, D = weight.shape
  B = x.size
  key = (V, D, B)
  if key not in _gather_cache:
    _gather_cache[key] = _make_gather(V, D, B)
  out = _gather_cache[key](x.reshape(-1).astype(jnp.int32), weight)
  return out.reshape(x.shape + (D,))
